# Initial kernel scaffold; baseline (speedup 1.0000x reference)
import jax
import jax.numpy as jnp
from jax.experimental import pallas as pl


def kernel(observation, executing_option, first_transition, W_m, b_m, W_mv, b_mv, W_t, b_t, W_a, b_a, W_v, b_v):
    raise NotImplementedError("write your pallas kernel here")


# speedup vs baseline: 1.7789x; 1.7789x over previous
import functools

import jax
import jax.numpy as jnp
from jax import lax
from jax.experimental import pallas as pl
from jax.experimental.pallas import tpu as pltpu

N = 16384
D = 768
E = 8
A = 8
W_COLS = 128
BLK = 1024

C_META = 0
C_MV = 8
C_TERM = 9
C_ACT = 17
C_VAL = 81
NEG = -3.0e38


def _body(obs_ref, w_ref, b_ref, opt_ref, first_ref,
          act_ref, val_ref, lp_ref, newopt_ref, mv_ref, mlp_ref, tp_ref):
    obs = obs_ref[...]
    h = jnp.dot(obs, w_ref[...], preferred_element_type=jnp.float32)
    h = h + b_ref[...]

    col = lax.broadcasted_iota(jnp.int32, (BLK, W_COLS), 1)
    opt = opt_ref[...]
    first = first_ref[...] != 0

    meta_mask = col < C_META + E
    hm = jnp.where(meta_mask, h, NEG)
    m_max = jnp.max(hm, axis=1, keepdims=True)
    m_arg = jnp.min(jnp.where(meta_mask & (h >= m_max), col, W_COLS),
                    axis=1, keepdims=True)
    m_sum = jnp.sum(jnp.where(meta_mask, jnp.exp(h - m_max), 0.0),
                    axis=1, keepdims=True)
    meta_log_probs = -jnp.log(m_sum)

    meta_values = jnp.sum(jnp.where(col == C_MV, h, 0.0), axis=1,
                          keepdims=True)

    t_logit = jnp.sum(jnp.where(col == C_TERM + opt, h, 0.0), axis=1,
                      keepdims=True)
    term_prob = 1.0 / (1.0 + jnp.exp(-t_logit))
    terminates = t_logit > 0.0

    requires_new = terminates | first
    new_opt = jnp.where(requires_new, m_arg, opt)
    term_prob = jnp.where(first, 0.0, term_prob)

    a_lo = C_ACT + new_opt * A
    act_mask = (col >= a_lo) & (col < a_lo + A)
    ha = jnp.where(act_mask, h, NEG)
    a_max = jnp.max(ha, axis=1, keepdims=True)
    a_arg = jnp.min(jnp.where(act_mask & (h >= a_max), col, W_COLS),
                    axis=1, keepdims=True) - a_lo
    a_sum = jnp.sum(jnp.where(act_mask, jnp.exp(h - a_max), 0.0),
                    axis=1, keepdims=True)
    log_probs = -jnp.log(a_sum)

    values = jnp.sum(jnp.where(col == C_VAL + new_opt, h, 0.0), axis=1,
                     keepdims=True)

    act_ref[...] = a_arg
    val_ref[...] = values
    lp_ref[...] = log_probs
    newopt_ref[...] = new_opt
    mv_ref[...] = meta_values
    mlp_ref[...] = meta_log_probs
    tp_ref[...] = term_prob


@jax.jit
def _run(observation, opt2d, first2d, w_cat, b_cat):
    grid = (N // BLK,)
    row_spec = pl.BlockSpec((BLK, 1), lambda i: (i, 0))
    f32 = jnp.float32
    outs = pl.pallas_call(
        _body,
        grid=grid,
        in_specs=[
            pl.BlockSpec((BLK, D), lambda i: (i, 0)),
            pl.BlockSpec((D, W_COLS), lambda i: (0, 0)),
            pl.BlockSpec((1, W_COLS), lambda i: (0, 0)),
            row_spec,
            row_spec,
        ],
        out_specs=[row_spec] * 7,
        out_shape=[
            jax.ShapeDtypeStruct((N, 1), jnp.int32),
            jax.ShapeDtypeStruct((N, 1), f32),
            jax.ShapeDtypeStruct((N, 1), f32),
            jax.ShapeDtypeStruct((N, 1), jnp.int32),
            jax.ShapeDtypeStruct((N, 1), f32),
            jax.ShapeDtypeStruct((N, 1), f32),
            jax.ShapeDtypeStruct((N, 1), f32),
        ],
        compiler_params=pltpu.CompilerParams(
            dimension_semantics=("arbitrary",),
        ),
    )(observation, w_cat, b_cat, opt2d, first2d)
    return outs


def kernel(observation, executing_option, first_transition,
           W_m, b_m, W_mv, b_mv, W_t, b_t, W_a, b_a, W_v, b_v):
    w_cat = jnp.concatenate(
        [
            W_m,
            W_mv,
            W_t,
            jnp.transpose(W_a, (1, 0, 2)).reshape(D, E * A),
            W_v[:, :, 0].T,
            jnp.zeros((D, W_COLS - (2 * E + 1 + E * A + E)),
                      dtype=jnp.float32),
        ],
        axis=1,
    )
    b_cat = jnp.concatenate(
        [b_m, b_mv, b_t, b_a.reshape(E * A), b_v[:, 0],
         jnp.zeros((W_COLS - (2 * E + 1 + E * A + E),), dtype=jnp.float32)],
    ).reshape(1, W_COLS)

    opt2d = executing_option.astype(jnp.int32).reshape(N, 1)
    first2d = first_transition.astype(jnp.int32).reshape(N, 1)

    (a2, v2, lp2, no2, mv2, mlp2, tp2) = _run(
        observation, opt2d, first2d, w_cat, b_cat)

    out_dtype = executing_option.dtype
    return (a2[:, 0], v2[:, 0], lp2[:, 0],
            no2[:, 0].astype(out_dtype), mv2[:, 0], mlp2[:, 0], tp2[:, 0])


# speedup vs baseline: 1.8448x; 1.0370x over previous
import functools

import jax
import jax.numpy as jnp
from jax import lax
from jax.experimental import pallas as pl
from jax.experimental.pallas import tpu as pltpu

N = 16384
D = 768
E = 8
A = 8
W_COLS = 128
BLK = 2048

C_META = 0
C_MV = 8
C_TERM = 9
C_ACT = 17
C_VAL = 81
NEG = -3.0e38


def _body(obs_ref, w_ref, b_ref, opt_ref, first_ref,
          act_ref, val_ref, lp_ref, newopt_ref, mv_ref, mlp_ref, tp_ref):
    obs = obs_ref[...]
    h = jnp.dot(obs, w_ref[...], preferred_element_type=jnp.float32)
    h = h + b_ref[...]

    col = lax.broadcasted_iota(jnp.int32, (BLK, W_COLS), 1)
    opt = opt_ref[...]
    first = first_ref[...] != 0

    meta_mask = col < C_META + E
    hm = jnp.where(meta_mask, h, NEG)
    m_max = jnp.max(hm, axis=1, keepdims=True)
    m_arg = jnp.min(jnp.where(meta_mask & (h >= m_max), col, W_COLS),
                    axis=1, keepdims=True)
    m_sum = jnp.sum(jnp.where(meta_mask, jnp.exp(h - m_max), 0.0),
                    axis=1, keepdims=True)
    meta_log_probs = -jnp.log(m_sum)

    meta_values = jnp.sum(jnp.where(col == C_MV, h, 0.0), axis=1,
                          keepdims=True)

    t_logit = jnp.sum(jnp.where(col == C_TERM + opt, h, 0.0), axis=1,
                      keepdims=True)
    term_prob = 1.0 / (1.0 + jnp.exp(-t_logit))
    terminates = t_logit > 0.0

    requires_new = terminates | first
    new_opt = jnp.where(requires_new, m_arg, opt)
    term_prob = jnp.where(first, 0.0, term_prob)

    a_lo = C_ACT + new_opt * A
    act_mask = (col >= a_lo) & (col < a_lo + A)
    ha = jnp.where(act_mask, h, NEG)
    a_max = jnp.max(ha, axis=1, keepdims=True)
    a_arg = jnp.min(jnp.where(act_mask & (h >= a_max), col, W_COLS),
                    axis=1, keepdims=True) - a_lo
    a_sum = jnp.sum(jnp.where(act_mask, jnp.exp(h - a_max), 0.0),
                    axis=1, keepdims=True)
    log_probs = -jnp.log(a_sum)

    values = jnp.sum(jnp.where(col == C_VAL + new_opt, h, 0.0), axis=1,
                     keepdims=True)

    act_ref[...] = a_arg
    val_ref[...] = values
    lp_ref[...] = log_probs
    newopt_ref[...] = new_opt
    mv_ref[...] = meta_values
    mlp_ref[...] = meta_log_probs
    tp_ref[...] = term_prob


@jax.jit
def _run(observation, opt2d, first2d, w_cat, b_cat):
    grid = (N // BLK,)
    row_spec = pl.BlockSpec((BLK, 1), lambda i: (i, 0))
    f32 = jnp.float32
    outs = pl.pallas_call(
        _body,
        grid=grid,
        in_specs=[
            pl.BlockSpec((BLK, D), lambda i: (i, 0)),
            pl.BlockSpec((D, W_COLS), lambda i: (0, 0)),
            pl.BlockSpec((1, W_COLS), lambda i: (0, 0)),
            row_spec,
            row_spec,
        ],
        out_specs=[row_spec] * 7,
        out_shape=[
            jax.ShapeDtypeStruct((N, 1), jnp.int32),
            jax.ShapeDtypeStruct((N, 1), f32),
            jax.ShapeDtypeStruct((N, 1), f32),
            jax.ShapeDtypeStruct((N, 1), jnp.int32),
            jax.ShapeDtypeStruct((N, 1), f32),
            jax.ShapeDtypeStruct((N, 1), f32),
            jax.ShapeDtypeStruct((N, 1), f32),
        ],
        compiler_params=pltpu.CompilerParams(
            dimension_semantics=("arbitrary",),
        ),
    )(observation, w_cat, b_cat, opt2d, first2d)
    return outs


def kernel(observation, executing_option, first_transition,
           W_m, b_m, W_mv, b_mv, W_t, b_t, W_a, b_a, W_v, b_v):
    w_cat = jnp.concatenate(
        [
            W_m,
            W_mv,
            W_t,
            jnp.transpose(W_a, (1, 0, 2)).reshape(D, E * A),
            W_v[:, :, 0].T,
            jnp.zeros((D, W_COLS - (2 * E + 1 + E * A + E)),
                      dtype=jnp.float32),
        ],
        axis=1,
    )
    b_cat = jnp.concatenate(
        [b_m, b_mv, b_t, b_a.reshape(E * A), b_v[:, 0],
         jnp.zeros((W_COLS - (2 * E + 1 + E * A + E),), dtype=jnp.float32)],
    ).reshape(1, W_COLS)

    opt2d = executing_option.astype(jnp.int32).reshape(N, 1)
    first2d = first_transition.astype(jnp.int32).reshape(N, 1)

    (a2, v2, lp2, no2, mv2, mlp2, tp2) = _run(
        observation, opt2d, first2d, w_cat, b_cat)

    out_dtype = executing_option.dtype
    return (a2[:, 0], v2[:, 0], lp2[:, 0],
            no2[:, 0].astype(out_dtype), mv2[:, 0], mlp2[:, 0], tp2[:, 0])


# speedup vs baseline: 5.1615x; 2.7979x over previous
import jax
import jax.numpy as jnp
from jax import lax
from jax.experimental import pallas as pl
from jax.experimental.pallas import tpu as pltpu

N = 16384
D = 768
E = 8
A = 8
W_ROWS = 128
BLK = 2048

R_META = 0
R_TERM = 8
R_VAL = 16
R_MV = 24
R_ACT = 32


def _body(obs_ref, wt_ref, b_ref, opt_ref, first_ref, out_ref):
    obs = obs_ref[...]
    h = lax.dot_general(wt_ref[...], obs, (((1,), (1,)), ((), ())),
                        preferred_element_type=jnp.float32)
    h = h + b_ref[...]

    opt = opt_ref[...]
    first = first_ref[...] != 0
    row8 = lax.broadcasted_iota(jnp.int32, (E, BLK), 0)

    meta = h[R_META:R_META + E, :]
    m_max = jnp.max(meta, axis=0, keepdims=True)
    m_arg = jnp.min(jnp.where(meta >= m_max, row8, E), axis=0, keepdims=True)
    m_sum = jnp.sum(jnp.exp(meta - m_max), axis=0, keepdims=True)
    meta_log_probs = -jnp.log(m_sum)

    meta_values = h[R_MV:R_MV + 1, :]

    term = h[R_TERM:R_TERM + E, :]
    t_logit = jnp.sum(jnp.where(row8 == opt, term, 0.0), axis=0,
                      keepdims=True)
    term_prob = 1.0 / (1.0 + jnp.exp(-t_logit))
    requires = (t_logit > 0.0) | first

    new_opt = jnp.where(requires, m_arg, opt)
    term_prob = jnp.where(first, 0.0, term_prob)

    vals = h[R_VAL:R_VAL + E, :]
    values = jnp.sum(jnp.where(row8 == new_opt, vals, 0.0), axis=0,
                     keepdims=True)

    sel = jnp.where(new_opt == 0, h[R_ACT:R_ACT + A, :], 0.0)
    for e in range(1, E):
        lo = R_ACT + A * e
        sel = sel + jnp.where(new_opt == e, h[lo:lo + A, :], 0.0)
    a_max = jnp.max(sel, axis=0, keepdims=True)
    a_arg = jnp.min(jnp.where(sel >= a_max, row8, E), axis=0, keepdims=True)
    a_sum = jnp.sum(jnp.exp(sel - a_max), axis=0, keepdims=True)
    log_probs = -jnp.log(a_sum)

    out_ref[...] = jnp.concatenate(
        [a_arg.astype(jnp.float32), values, log_probs,
         new_opt.astype(jnp.float32), meta_values, meta_log_probs,
         term_prob, jnp.zeros_like(term_prob)], axis=0)


@jax.jit
def _run(observation, opt1, first1, wt, b_col):
    return pl.pallas_call(
        _body,
        grid=(N // BLK,),
        in_specs=[
            pl.BlockSpec((BLK, D), lambda i: (i, 0)),
            pl.BlockSpec((W_ROWS, D), lambda i: (0, 0)),
            pl.BlockSpec((W_ROWS, 1), lambda i: (0, 0)),
            pl.BlockSpec((1, BLK), lambda i: (0, i)),
            pl.BlockSpec((1, BLK), lambda i: (0, i)),
        ],
        out_specs=pl.BlockSpec((8, BLK), lambda i: (0, i)),
        out_shape=jax.ShapeDtypeStruct((8, N), jnp.float32),
        compiler_params=pltpu.CompilerParams(
            dimension_semantics=("arbitrary",),
        ),
    )(observation, wt, b_col, opt1, first1)


def kernel(observation, executing_option, first_transition,
           W_m, b_m, W_mv, b_mv, W_t, b_t, W_a, b_a, W_v, b_v):
    pad_rows = W_ROWS - R_ACT - E * A
    wt = jnp.concatenate(
        [
            W_m.T,
            W_t.T,
            W_v[:, :, 0],
            W_mv.T,
            jnp.zeros((R_ACT - R_MV - 1, D), jnp.float32),
            jnp.transpose(W_a, (0, 2, 1)).reshape(E * A, D),
            jnp.zeros((pad_rows, D), jnp.float32),
        ],
        axis=0,
    )
    b_col = jnp.concatenate(
        [b_m, b_t, b_v[:, 0], b_mv,
         jnp.zeros((R_ACT - R_MV - 1,), jnp.float32),
         b_a.reshape(E * A),
         jnp.zeros((pad_rows,), jnp.float32)],
    ).reshape(W_ROWS, 1)

    opt1 = executing_option.astype(jnp.int32).reshape(1, N)
    first1 = first_transition.astype(jnp.int32).reshape(1, N)

    out = _run(observation, opt1, first1, wt, b_col)

    out_dtype = executing_option.dtype
    return (out[0].astype(jnp.int32), out[1], out[2],
            out[3].astype(out_dtype), out[4], out[5], out[6])
